# baseline (speedup 1.0000x reference)
import functools

import jax
import jax.numpy as jnp
from jax.experimental import pallas as pl
from jax.experimental.pallas import tpu as pltpu

Q = 1024
D = 16
K = 100000
KB = 2048
NBLK = (K + KB - 1) // KB
KPAD = NBLK * KB

_BIG = 1e30


def _butterfly(s, axis):
    while s.shape[axis] > 1:
        h = s.shape[axis] // 2
        if axis == 0:
            s = s[:h, :] + s[h:, :]
        else:
            s = s[:, :h] + s[:, h:]
    return s


def _ksq_kernel(k_ref, out_ref):
    j = pl.program_id(0)
    kt = k_ref[...].T
    ksq = _butterfly(kt * kt, axis=0)
    col = jax.lax.broadcasted_iota(jnp.int32, (1, KB), 1) + j * KB
    out_ref[...] = jnp.where(col < K, ksq, jnp.float32(_BIG))


def _knn_kernel(qt_ref, k_ref, ksq_ref, scores_ref, idx_ref, img_ref,
                min_s, idx_s, qsq_s):
    j = pl.program_id(0)
    nblk = pl.num_programs(0)

    @pl.when(j == 0)
    def _init():
        min_s[...] = jnp.full((1, Q), jnp.float32(_BIG))
        idx_s[...] = jnp.zeros((1, Q), dtype=jnp.int32)
        qt0 = qt_ref[...]
        qsq_s[...] = _butterfly(qt0 * qt0, axis=0)

    k = k_ref[...]
    k_sq = ksq_ref[...]

    qk2 = jax.lax.dot_general((k * 2.0).astype(jnp.bfloat16),
                              qt_ref[...].astype(jnp.bfloat16),
                              (((1,), (0,)), ((), ())),
                              preferred_element_type=jnp.float32)
    d = (k_sq + qsq_s[...]) - qk2

    bmin = jnp.min(d, axis=0, keepdims=True)
    bidx = jnp.argmin(d, axis=0).astype(jnp.int32)[None, :] + j * KB

    run_min = min_s[...]
    upd = bmin < run_min
    min_s[...] = jnp.where(upd, bmin, run_min)
    idx_s[...] = jnp.where(upd, bidx, idx_s[...])

    @pl.when(j == nblk - 1)
    def _fin():
        final = min_s[...]
        scores_ref[...] = final
        idx_ref[...] = idx_s[...]
        img_ref[...] = jnp.max(final).reshape(1, 1)


@functools.partial(jax.jit, static_argnames=())
def _knn(queries_t, keys):
    ksq_row = pl.pallas_call(
        _ksq_kernel,
        grid=(NBLK,),
        in_specs=[pl.BlockSpec((KB, D), lambda j: (j, 0))],
        out_specs=pl.BlockSpec((1, KB), lambda j: (0, j)),
        out_shape=jax.ShapeDtypeStruct((1, KPAD), jnp.float32),
        compiler_params=pltpu.CompilerParams(
            dimension_semantics=("arbitrary",),
        ),
    )(keys)
    ksq_col = ksq_row.reshape(KPAD, 1)
    return pl.pallas_call(
        _knn_kernel,
        grid=(NBLK,),
        in_specs=[
            pl.BlockSpec((D, Q), lambda j: (0, 0)),
            pl.BlockSpec((KB, D), lambda j: (j, 0)),
            pl.BlockSpec((KB, 1), lambda j: (j, 0)),
        ],
        out_specs=[
            pl.BlockSpec((1, Q), lambda j: (0, 0)),
            pl.BlockSpec((1, Q), lambda j: (0, 0)),
            pl.BlockSpec((1, 1), lambda j: (0, 0)),
        ],
        out_shape=[
            jax.ShapeDtypeStruct((1, Q), jnp.float32),
            jax.ShapeDtypeStruct((1, Q), jnp.int32),
            jax.ShapeDtypeStruct((1, 1), jnp.float32),
        ],
        scratch_shapes=[
            pltpu.VMEM((1, Q), jnp.float32),
            pltpu.VMEM((1, Q), jnp.int32),
            pltpu.VMEM((1, Q), jnp.float32),
        ],
        compiler_params=pltpu.CompilerParams(
            dimension_semantics=("arbitrary",),
        ),
    )(queries_t, keys, ksq_col)


def kernel(queries, keys):
    scores, idx, img = _knn(queries.T, keys)
    patch_scores = scores.reshape(Q)
    image_score = img.reshape(())
    nn_idx = idx.reshape(Q, 1)
    return patch_scores, image_score, nn_idx


# speedup vs baseline: 1.2657x; 1.2657x over previous
import functools

import jax
import jax.numpy as jnp
from jax.experimental import pallas as pl
from jax.experimental.pallas import tpu as pltpu

Q = 1024
D = 16
K = 100000
KB = 2048
NBLK = (K + KB - 1) // KB

_BIG = 1e30


def _butterfly(s, axis):
    while s.shape[axis] > 1:
        h = s.shape[axis] // 2
        if axis == 0:
            s = s[:h, :] + s[h:, :]
        else:
            s = s[:, :h] + s[:, h:]
    return s


def _knn_kernel(qt_ref, k_ref, scores_ref, idx_ref, img_ref,
                min_s, idx_s, qsq_s):
    j = pl.program_id(0)
    nblk = pl.num_programs(0)

    @pl.when(j == 0)
    def _init():
        min_s[...] = jnp.full((1, Q), jnp.float32(_BIG))
        idx_s[...] = jnp.zeros((1, Q), dtype=jnp.int32)
        qt0 = qt_ref[...]
        qsq_s[...] = _butterfly(qt0 * qt0, axis=0)

    k = k_ref[...]
    kt = k.T
    ksq_row = _butterfly(kt * kt, axis=0)
    col = jax.lax.broadcasted_iota(jnp.int32, (1, KB), 1) + j * KB
    ksq_row = jnp.where(col < K, ksq_row, jnp.float32(_BIG))
    k_sq = ksq_row.T

    qk2 = jax.lax.dot_general(k.astype(jnp.bfloat16),
                              (qt_ref[...] * 2.0).astype(jnp.bfloat16),
                              (((1,), (0,)), ((), ())),
                              preferred_element_type=jnp.float32)
    d = (k_sq + qsq_s[...]) - qk2

    bmin = jnp.min(d, axis=0, keepdims=True)
    bidx = jnp.argmin(d, axis=0).astype(jnp.int32)[None, :] + j * KB

    run_min = min_s[...]
    upd = bmin < run_min
    min_s[...] = jnp.where(upd, bmin, run_min)
    idx_s[...] = jnp.where(upd, bidx, idx_s[...])

    @pl.when(j == nblk - 1)
    def _fin():
        final = min_s[...]
        scores_ref[...] = final
        idx_ref[...] = idx_s[...]
        img_ref[...] = jnp.max(final).reshape(1, 1)


@functools.partial(jax.jit, static_argnames=())
def _knn(queries_t, keys):
    return pl.pallas_call(
        _knn_kernel,
        grid=(NBLK,),
        in_specs=[
            pl.BlockSpec((D, Q), lambda j: (0, 0)),
            pl.BlockSpec((KB, D), lambda j: (j, 0)),
        ],
        out_specs=[
            pl.BlockSpec((1, Q), lambda j: (0, 0)),
            pl.BlockSpec((1, Q), lambda j: (0, 0)),
            pl.BlockSpec((1, 1), lambda j: (0, 0)),
        ],
        out_shape=[
            jax.ShapeDtypeStruct((1, Q), jnp.float32),
            jax.ShapeDtypeStruct((1, Q), jnp.int32),
            jax.ShapeDtypeStruct((1, 1), jnp.float32),
        ],
        scratch_shapes=[
            pltpu.VMEM((1, Q), jnp.float32),
            pltpu.VMEM((1, Q), jnp.int32),
            pltpu.VMEM((1, Q), jnp.float32),
        ],
        compiler_params=pltpu.CompilerParams(
            dimension_semantics=("arbitrary",),
        ),
    )(queries_t, keys)


def kernel(queries, keys):
    scores, idx, img = _knn(queries.T, keys)
    patch_scores = scores.reshape(Q)
    image_score = img.reshape(())
    nn_idx = idx.reshape(Q, 1)
    return patch_scores, image_score, nn_idx
